# baseline (speedup 1.0000x reference)
import functools

import jax
import jax.numpy as jnp
import numpy as np
from jax import lax
from jax.experimental import pallas as pl
from jax.experimental.pallas import tpu as pltpu
from jax.experimental.pallas import tpu_sc as plsc

N = 1024
L = 32768
B = 32
K = 8
THR = 10.0

N1 = 128
N2 = 256
KH = 136


def _sc_knn_kernel(qs_h, px_h, py_h, pz_h, outk_h, outi_h,
                   px_v, py_v, pz_v, q3_v, rowk_v, rowi_v):
    wid = lax.axis_index("s") * 2 + lax.axis_index("c")
    pltpu.sync_copy(px_h, px_v)
    pltpu.sync_copy(py_h, py_v)
    pltpu.sync_copy(pz_h, pz_v)

    lane = lax.iota(jnp.int32, 16)

    def one_query(q):
        pltpu.sync_copy(qs_h.at[q], q3_v)
        qxv = q3_v[pl.ds(0, 16)]
        qyv = q3_v[pl.ds(16, 16)]
        qzv = q3_v[pl.ds(32, 16)]

        def body(c, carry):
            keys = list(carry[:K])
            idxs = list(carry[K:])
            base = c * 16
            dx = px_v[pl.ds(base, 16)] - qxv
            dy = py_v[pl.ds(base, 16)] - qyv
            dz = pz_v[pl.ds(base, 16)] - qzv
            v = dx * dx + dy * dy + dz * dz
            vi = lane + base
            for r in range(K):
                lt = v < keys[r]
                nk = jnp.where(lt, v, keys[r])
                ni = jnp.where(lt, vi, idxs[r])
                v = jnp.where(lt, keys[r], v)
                vi = jnp.where(lt, idxs[r], vi)
                keys[r] = nk
                idxs[r] = ni
            return tuple(keys) + tuple(idxs)

        init = (tuple(jnp.full((16,), jnp.inf, jnp.float32)
                      for _ in range(K)) +
                tuple(jnp.full((16,), np.int32(2**30)) for _ in range(K)))
        carry = lax.fori_loop(0, N // 16, body, init)
        for r in range(K):
            rowk_v[pl.ds(r * 16, 16)] = carry[r]
            rowi_v[pl.ds(r * 16, 16)] = carry[K + r]
        pltpu.sync_copy(rowk_v, outk_h.at[q])
        pltpu.sync_copy(rowi_v, outi_h.at[q])

    one_query(wid * 2)
    one_query(wid * 2 + 1)


def _merge_kernel(d2_ref, idx_ref, out_ref):
    s = jnp.sqrt(d2_ref[...])
    cidx = idx_ref[...]
    big = np.int32(2**30)
    for k in range(K):
        m = jnp.min(s, axis=1, keepdims=True)
        idxk = jnp.min(jnp.where(s == m, cidx, big), axis=1, keepdims=True)
        out_ref[:, k:k + 1] = idxk
        s = jnp.where(cidx == idxk, jnp.inf, s)


def _sc_topk_indices(input_posA, input_posB, positions):
    q = jnp.concatenate([input_posA, input_posB], axis=0)
    qs = jnp.concatenate([jnp.broadcast_to(q[:, c:c + 1], (2 * B, 16))
                          for c in range(3)], axis=1)
    mesh = plsc.VectorSubcoreMesh(core_axis_name="c", subcore_axis_name="s")
    fn = functools.partial(
        pl.kernel, mesh=mesh,
        out_type=[
            jax.ShapeDtypeStruct((2 * B, K * 16), jnp.float32),
            jax.ShapeDtypeStruct((2 * B, K * 16), jnp.int32),
        ],
        scratch_types=[
            pltpu.VMEM((N,), jnp.float32),
            pltpu.VMEM((N,), jnp.float32),
            pltpu.VMEM((N,), jnp.float32),
            pltpu.VMEM((48,), jnp.float32),
            pltpu.VMEM((K * 16,), jnp.float32),
            pltpu.VMEM((K * 16,), jnp.int32),
        ],
    )(_sc_knn_kernel)
    d2c, idxc = fn(qs, positions[:, 0], positions[:, 1], positions[:, 2])
    return pl.pallas_call(
        _merge_kernel,
        out_shape=jax.ShapeDtypeStruct((2 * B, K), jnp.int32),
    )(d2c, idxc)


def _split(x):
    hi = x.astype(jnp.bfloat16)
    lo = (x - hi.astype(jnp.float32)).astype(jnp.bfloat16)
    return hi, lo


def _main_kernel(idx_ref, *refs):
    audio_refs = refs[:2 * K]
    (f2rh_ref, f2rl_ref, f2ih_ref, f2il_ref, wr_ref, wi_ref,
     f1rh_ref, f1rl_ref, f1ih_ref, f1il_ref, rrev_ref, csel_ref,
     outr_ref, outi_ref) = refs[2 * K:]

    x2 = jnp.concatenate([r[0] for r in audio_refs], axis=1)

    dot = functools.partial(
        jax.lax.dot_general,
        preferred_element_type=jnp.float32)
    dnum = (((1,), (0,)), ((), ()))

    def dot3(ah, al, bh, bl):
        return (dot(ah, bh, dnum) + dot(ah, bl, dnum) + dot(al, bh, dnum))

    xh, xl = _split(x2)
    gr = dot3(f2rh_ref[...], f2rl_ref[...], xh, xl)
    gi = dot3(f2ih_ref[...], f2il_ref[...], xh, xl)

    zr = gr * wr_ref[...] - gi * wi_ref[...]
    zi = gr * wi_ref[...] + gi * wr_ref[...]

    zr = zr.reshape(KH * 2 * K, N1)
    zi = zi.reshape(KH * 2 * K, N1)
    zrh, zrl = _split(zr)
    zih, zil = _split(zi)
    xr = (dot3(zrh, zrl, f1rh_ref[...], f1rl_ref[...]) -
          dot3(zih, zil, f1ih_ref[...], f1il_ref[...]))
    xi = (dot3(zrh, zrl, f1ih_ref[...], f1il_ref[...]) +
          dot3(zih, zil, f1rh_ref[...], f1rl_ref[...]))
    xr = xr.reshape(KH, 2 * K, N1)
    xi = xi.reshape(KH, 2 * K, N1)

    ar = xr[:, :K, :]
    ai = xi[:, :K, :]
    br = xr[:, K:, :]
    bi = xi[:, K:, :]

    recip = 1.0 / (ar * ar + ai * ai)
    acc_r = jnp.zeros((KH, K, N1), dtype=jnp.float32)
    acc_i = jnp.zeros((KH, K, N1), dtype=jnp.float32)
    for s in range(K):
        if s == 0:
            brs, bis = br, bi
        else:
            brs = jnp.concatenate([br[:, s:, :], br[:, :s, :]], axis=1)
            bis = jnp.concatenate([bi[:, s:, :], bi[:, :s, :]], axis=1)
        num_r = brs * ar + bis * ai
        num_i = bis * ar - brs * ai
        rr = num_r * recip
        ri = num_i * recip
        rr = jnp.where(jnp.isnan(rr), 0.0, rr)
        ri = jnp.where(jnp.isnan(ri), 0.0, ri)
        acc_r = acc_r + jnp.clip(rr, -THR, THR)
        acc_i = acc_i + jnp.clip(ri, -THR, THR)

    res_r = (jnp.sum(acc_r, axis=1) * (1.0 / (K * K))).T
    res_i = (jnp.sum(acc_i, axis=1) * (1.0 / (K * K))).T
    doth = functools.partial(jax.lax.dot_general,
                             precision=jax.lax.Precision.HIGHEST,
                             preferred_element_type=jnp.float32)
    mir_r = doth(doth(rrev_ref[...], res_r, dnum), csel_ref[...], dnum)
    mir_i = doth(doth(rrev_ref[...], res_i, dnum), csel_ref[...], dnum)
    outr_ref[0] = jnp.concatenate([res_r[:, :129], mir_r], axis=1)
    outi_ref[0] = jnp.concatenate([res_i[:, :129], -mir_i], axis=1)


def _dft_tables():
    k2 = np.arange(KH)
    n2 = np.arange(N2)
    ang = (-2.0 * np.pi / N2) * (np.outer(k2, n2) % N2)
    f2r = np.cos(ang).astype(np.float32)
    f2i = np.sin(ang).astype(np.float32)
    n1 = np.arange(N1)
    angw = (-2.0 * np.pi / L) * (np.outer(k2, n1) % L)
    wr = np.tile(np.cos(angw).astype(np.float32), (1, 2 * K))
    wi = np.tile(np.sin(angw).astype(np.float32), (1, 2 * K))
    ang1 = (-2.0 * np.pi / N1) * (np.outer(n1, n1) % N1)
    f1r = np.cos(ang1).astype(np.float32)
    f1i = np.sin(ang1).astype(np.float32)

    def split(a):
        hi = a.astype(jnp.bfloat16)
        lo = (a - np.asarray(hi, np.float32)).astype(jnp.bfloat16)
        return jnp.asarray(hi), jnp.asarray(lo)

    rrev = np.eye(N1, dtype=np.float32)[::-1]
    csel = np.zeros((KH, 127), dtype=np.float32)
    for j in range(127):
        csel[127 - j, j] = 1.0

    return (*split(f2r), *split(f2i), jnp.asarray(wr), jnp.asarray(wi),
            *split(f1r), *split(f1i), jnp.asarray(rrev), jnp.asarray(csel))


def kernel(input_posA, input_posB, positions, audios):
    idx = _sc_topk_indices(input_posA, input_posB, positions)
    idx_flat = idx.reshape(-1)

    audios3 = audios.reshape(N, N2, N1)
    (f2rh, f2rl, f2ih, f2il, wr, wi,
     f1rh, f1rl, f1ih, f1il, rrev, csel) = _dft_tables()

    def audio_spec(t):
        if t < K:
            def imap(b, iref, t=t):
                return (iref[b * K + t], 0, 0)
        else:
            def imap(b, iref, t=t):
                return (iref[B * K + b * K + (t - K)], 0, 0)
        return pl.BlockSpec((1, N2, N1), imap)

    in_specs = [audio_spec(t) for t in range(2 * K)]
    in_specs += [pl.BlockSpec((KH, N2), lambda b, iref: (0, 0))] * 4
    in_specs += [pl.BlockSpec((KH, 2 * K * N1), lambda b, iref: (0, 0))] * 2
    in_specs += [pl.BlockSpec((N1, N1), lambda b, iref: (0, 0))] * 4
    in_specs += [
        pl.BlockSpec((N1, N1), lambda b, iref: (0, 0)),
        pl.BlockSpec((KH, 127), lambda b, iref: (0, 0)),
    ]
    out_specs = [
        pl.BlockSpec((1, N1, N2), lambda b, iref: (b, 0, 0)),
        pl.BlockSpec((1, N1, N2), lambda b, iref: (b, 0, 0)),
    ]

    outr, outi = pl.pallas_call(
        _main_kernel,
        grid_spec=pltpu.PrefetchScalarGridSpec(
            num_scalar_prefetch=1,
            grid=(B,),
            in_specs=in_specs,
            out_specs=out_specs,
        ),
        out_shape=[
            jax.ShapeDtypeStruct((B, N1, N2), jnp.float32),
            jax.ShapeDtypeStruct((B, N1, N2), jnp.float32),
        ],
    )(idx_flat, *([audios3] * (2 * K)), f2rh, f2rl, f2ih, f2il,
      wr, wi, f1rh, f1rl, f1ih, f1il, rrev, csel)

    return (outr + 1j * outi).reshape(B, L)


# speedup vs baseline: 1.0410x; 1.0410x over previous
import functools

import jax
import jax.numpy as jnp
import numpy as np
from jax import lax
from jax.experimental import pallas as pl
from jax.experimental.pallas import tpu as pltpu
from jax.experimental.pallas import tpu_sc as plsc

N = 1024
L = 32768
B = 32
K = 8
THR = 10.0

N1 = 128
N2 = 256
KH = 136


def _sc_knn_kernel(qs_h, px_h, py_h, pz_h, outk_h, outi_h,
                   px_v, py_v, pz_v, q3_v, rowk_v, rowi_v):
    wid = lax.axis_index("s") * 2 + lax.axis_index("c")
    pltpu.sync_copy(px_h, px_v)
    pltpu.sync_copy(py_h, py_v)
    pltpu.sync_copy(pz_h, pz_v)

    lane = lax.iota(jnp.int32, 16)

    def one_query(q):
        pltpu.sync_copy(qs_h.at[q], q3_v)
        qxv = q3_v[pl.ds(0, 16)]
        qyv = q3_v[pl.ds(16, 16)]
        qzv = q3_v[pl.ds(32, 16)]

        def body(c, carry):
            keys = list(carry[:K])
            idxs = list(carry[K:])
            base = c * 16
            dx = px_v[pl.ds(base, 16)] - qxv
            dy = py_v[pl.ds(base, 16)] - qyv
            dz = pz_v[pl.ds(base, 16)] - qzv
            v = dx * dx + dy * dy + dz * dz
            vi = lane + base
            for r in range(K):
                lt = v < keys[r]
                nk = jnp.where(lt, v, keys[r])
                ni = jnp.where(lt, vi, idxs[r])
                v = jnp.where(lt, keys[r], v)
                vi = jnp.where(lt, idxs[r], vi)
                keys[r] = nk
                idxs[r] = ni
            return tuple(keys) + tuple(idxs)

        init = (tuple(jnp.full((16,), jnp.inf, jnp.float32)
                      for _ in range(K)) +
                tuple(jnp.full((16,), np.int32(2**30)) for _ in range(K)))
        carry = lax.fori_loop(0, N // 16, body, init)
        for r in range(K):
            rowk_v[pl.ds(r * 16, 16)] = carry[r]
            rowi_v[pl.ds(r * 16, 16)] = carry[K + r]
        pltpu.sync_copy(rowk_v, outk_h.at[q])
        pltpu.sync_copy(rowi_v, outi_h.at[q])

    one_query(wid * 2)
    one_query(wid * 2 + 1)


def _merge_kernel(d2_ref, idx_ref, out_ref):
    s = jnp.sqrt(d2_ref[...])
    cidx = idx_ref[...]
    big = np.int32(2**30)
    for k in range(K):
        m = jnp.min(s, axis=1, keepdims=True)
        idxk = jnp.min(jnp.where(s == m, cidx, big), axis=1, keepdims=True)
        out_ref[:, k:k + 1] = idxk
        s = jnp.where(cidx == idxk, jnp.inf, s)


def _sc_topk_indices(input_posA, input_posB, positions):
    q = jnp.concatenate([input_posA, input_posB], axis=0)
    qs = jnp.concatenate([jnp.broadcast_to(q[:, c:c + 1], (2 * B, 16))
                          for c in range(3)], axis=1)
    mesh = plsc.VectorSubcoreMesh(core_axis_name="c", subcore_axis_name="s")
    fn = functools.partial(
        pl.kernel, mesh=mesh,
        out_type=[
            jax.ShapeDtypeStruct((2 * B, K * 16), jnp.float32),
            jax.ShapeDtypeStruct((2 * B, K * 16), jnp.int32),
        ],
        scratch_types=[
            pltpu.VMEM((N,), jnp.float32),
            pltpu.VMEM((N,), jnp.float32),
            pltpu.VMEM((N,), jnp.float32),
            pltpu.VMEM((48,), jnp.float32),
            pltpu.VMEM((K * 16,), jnp.float32),
            pltpu.VMEM((K * 16,), jnp.int32),
        ],
    )(_sc_knn_kernel)
    d2c, idxc = fn(qs, positions[:, 0], positions[:, 1], positions[:, 2])
    return pl.pallas_call(
        _merge_kernel,
        out_shape=jax.ShapeDtypeStruct((2 * B, K), jnp.int32),
    )(d2c, idxc)


def _split(x):
    hi = x.astype(jnp.bfloat16)
    lo = (x - hi.astype(jnp.float32)).astype(jnp.bfloat16)
    return hi, lo


def _main_kernel(idx_ref, *refs):
    audio_refs = refs[:2 * K]
    (f2rc_ref, f2ic_ref, wr_ref, wi_ref, f1rc_ref, f1ic_ref,
     rrev_ref, csel_ref, outr_ref, outi_ref) = refs[2 * K:]

    x2 = jnp.concatenate([r[0] for r in audio_refs], axis=1)

    dot = functools.partial(
        jax.lax.dot_general,
        preferred_element_type=jnp.float32)
    dnum = (((1,), (0,)), ((), ()))


    xh, xl = _split(x2)
    xcat = jnp.concatenate([xh, xl, xh], axis=0)
    gr = dot(f2rc_ref[...], xcat, dnum)
    gi = dot(f2ic_ref[...], xcat, dnum)

    zr = gr * wr_ref[...] - gi * wi_ref[...]
    zi = gr * wi_ref[...] + gi * wr_ref[...]

    zr = zr.reshape(KH * 2 * K, N1)
    zi = zi.reshape(KH * 2 * K, N1)
    zrh, zrl = _split(zr)
    zih, zil = _split(zi)
    zrcat = jnp.concatenate([zrh, zrh, zrl], axis=1)
    zicat = jnp.concatenate([zih, zih, zil], axis=1)
    xr = dot(zrcat, f1rc_ref[...], dnum) - dot(zicat, f1ic_ref[...], dnum)
    xi = dot(zrcat, f1ic_ref[...], dnum) + dot(zicat, f1rc_ref[...], dnum)
    xr = xr.reshape(KH, 2 * K, N1)
    xi = xi.reshape(KH, 2 * K, N1)

    ar = xr[:, :K, :]
    ai = xi[:, :K, :]
    br = xr[:, K:, :]
    bi = xi[:, K:, :]

    recip = 1.0 / (ar * ar + ai * ai)
    acc_r = jnp.zeros((KH, K, N1), dtype=jnp.float32)
    acc_i = jnp.zeros((KH, K, N1), dtype=jnp.float32)
    for s in range(K):
        if s == 0:
            brs, bis = br, bi
        else:
            brs = jnp.concatenate([br[:, s:, :], br[:, :s, :]], axis=1)
            bis = jnp.concatenate([bi[:, s:, :], bi[:, :s, :]], axis=1)
        num_r = brs * ar + bis * ai
        num_i = bis * ar - brs * ai
        rr = num_r * recip
        ri = num_i * recip
        rr = jnp.where(jnp.isnan(rr), 0.0, rr)
        ri = jnp.where(jnp.isnan(ri), 0.0, ri)
        acc_r = acc_r + jnp.clip(rr, -THR, THR)
        acc_i = acc_i + jnp.clip(ri, -THR, THR)

    res_r = (jnp.sum(acc_r, axis=1) * (1.0 / (K * K))).T
    res_i = (jnp.sum(acc_i, axis=1) * (1.0 / (K * K))).T
    doth = functools.partial(jax.lax.dot_general,
                             precision=jax.lax.Precision.HIGHEST,
                             preferred_element_type=jnp.float32)
    mir_r = doth(doth(rrev_ref[...], res_r, dnum), csel_ref[...], dnum)
    mir_i = doth(doth(rrev_ref[...], res_i, dnum), csel_ref[...], dnum)
    outr_ref[0] = jnp.concatenate([res_r[:, :129], mir_r], axis=1)
    outi_ref[0] = jnp.concatenate([res_i[:, :129], -mir_i], axis=1)


def _dft_tables():
    k2 = np.arange(KH)
    n2 = np.arange(N2)
    ang = (-2.0 * np.pi / N2) * (np.outer(k2, n2) % N2)
    f2r = np.cos(ang).astype(np.float32)
    f2i = np.sin(ang).astype(np.float32)
    n1 = np.arange(N1)
    angw = (-2.0 * np.pi / L) * (np.outer(k2, n1) % L)
    wr = np.tile(np.cos(angw).astype(np.float32), (1, 2 * K))
    wi = np.tile(np.sin(angw).astype(np.float32), (1, 2 * K))
    ang1 = (-2.0 * np.pi / N1) * (np.outer(n1, n1) % N1)
    f1r = np.cos(ang1).astype(np.float32)
    f1i = np.sin(ang1).astype(np.float32)

    def split(a):
        hi = a.astype(jnp.bfloat16)
        lo = (a - np.asarray(hi, np.float32)).astype(jnp.bfloat16)
        return np.asarray(hi), np.asarray(lo)

    rrev = np.eye(N1, dtype=np.float32)[::-1]
    csel = np.zeros((KH, 127), dtype=np.float32)
    for j in range(127):
        csel[127 - j, j] = 1.0

    f2rh, f2rl = split(f2r)
    f2ih, f2il = split(f2i)
    f2rc = np.concatenate([f2rh, f2rh, f2rl], axis=1)
    f2ic = np.concatenate([f2ih, f2ih, f2il], axis=1)
    f1rh, f1rl = split(f1r)
    f1ih, f1il = split(f1i)
    f1rc = np.concatenate([f1rh, f1rl, f1rh], axis=0)
    f1ic = np.concatenate([f1ih, f1il, f1ih], axis=0)

    return (jnp.asarray(f2rc), jnp.asarray(f2ic), jnp.asarray(wr),
            jnp.asarray(wi), jnp.asarray(f1rc), jnp.asarray(f1ic),
            jnp.asarray(rrev), jnp.asarray(csel))


def kernel(input_posA, input_posB, positions, audios):
    idx = _sc_topk_indices(input_posA, input_posB, positions)
    idx_flat = idx.reshape(-1)

    audios3 = audios.reshape(N, N2, N1)
    f2rc, f2ic, wr, wi, f1rc, f1ic, rrev, csel = _dft_tables()

    def audio_spec(t):
        if t < K:
            def imap(b, iref, t=t):
                return (iref[b * K + t], 0, 0)
        else:
            def imap(b, iref, t=t):
                return (iref[B * K + b * K + (t - K)], 0, 0)
        return pl.BlockSpec((1, N2, N1), imap)

    in_specs = [audio_spec(t) for t in range(2 * K)]
    in_specs += [pl.BlockSpec((KH, 3 * N2), lambda b, iref: (0, 0))] * 2
    in_specs += [pl.BlockSpec((KH, 2 * K * N1), lambda b, iref: (0, 0))] * 2
    in_specs += [pl.BlockSpec((3 * N1, N1), lambda b, iref: (0, 0))] * 2
    in_specs += [
        pl.BlockSpec((N1, N1), lambda b, iref: (0, 0)),
        pl.BlockSpec((KH, 127), lambda b, iref: (0, 0)),
    ]
    out_specs = [
        pl.BlockSpec((1, N1, N2), lambda b, iref: (b, 0, 0)),
        pl.BlockSpec((1, N1, N2), lambda b, iref: (b, 0, 0)),
    ]

    outr, outi = pl.pallas_call(
        _main_kernel,
        grid_spec=pltpu.PrefetchScalarGridSpec(
            num_scalar_prefetch=1,
            grid=(B,),
            in_specs=in_specs,
            out_specs=out_specs,
        ),
        out_shape=[
            jax.ShapeDtypeStruct((B, N1, N2), jnp.float32),
            jax.ShapeDtypeStruct((B, N1, N2), jnp.float32),
        ],
    )(idx_flat, *([audios3] * (2 * K)), f2rc, f2ic,
      wr, wi, f1rc, f1ic, rrev, csel)

    return (outr + 1j * outi).reshape(B, L)
